# Initial kernel scaffold; baseline (speedup 1.0000x reference)
import jax
import jax.numpy as jnp
from jax.experimental import pallas as pl


def kernel(scores):
    raise NotImplementedError("write your pallas kernel here")


# speedup vs baseline: 3.2371x; 3.2371x over previous
import functools

import jax
import jax.numpy as jnp
from jax import lax
from jax.experimental import pallas as pl
from jax.experimental.pallas import tpu as pltpu
from jax.experimental.pallas import tpu_sc as plsc

ROWS = 64
N = 1_000_000
K = 1024
CHUNK = 10_000
NCHUNK = N // CHUNK
VPC = CHUNK // 16
CAP = 2048
CKLEN = CAP + 16
ROWS_PER_W = 2
_INT_MIN = -2147483648


def _skey(u):
    m = lax.shift_right_arithmetic(u, 31)
    return u ^ lax.shift_right_logical(m, 1)


def _topk_body(scores, vals_out, idx_out, buf0, buf1, ck, ci, ckb, cib,
               hist, h3, vstage, istage, sem0, sem1):
    iota = lax.iota(jnp.int32, 16)
    ones = jnp.ones((16,), jnp.int32)
    lane4096 = iota * 4096
    lane128 = iota * 128

    def s_at(v, i):
        return lax.squeeze(lax.slice(v, (i,), (i + 1,)), (0,))

    def splat(x):
        return lax.broadcast(x, (16,))

    wid = lax.axis_index("s") * 2 + lax.axis_index("c")

    def stream_row(row, process, carry0):
        pltpu.async_copy(scores.at[row, pl.ds(0, CHUNK)], buf0, sem0)

        def outer(i, carry):
            c0 = 2 * i
            cp1 = pltpu.async_copy(
                scores.at[row, pl.ds((c0 + 1) * CHUNK, CHUNK)], buf1, sem1)
            pltpu.make_async_copy(
                scores.at[row, pl.ds(c0 * CHUNK, CHUNK)], buf0, sem0).wait()
            carry = process(c0, buf0, carry)

            @pl.when(i < NCHUNK // 2 - 1)
            def _():
                pltpu.async_copy(
                    scores.at[row, pl.ds((c0 + 2) * CHUNK, CHUNK)], buf0, sem0)

            cp1.wait()
            carry = process(c0 + 1, buf1, carry)
            return carry

        return lax.fori_loop(0, NCHUNK // 2, outer, carry0)

    def do_row(row):
        def clr(j, c):
            hist[pl.ds(j * 16, 16)] = jnp.zeros((16,), jnp.int32)
            return c

        lax.fori_loop(0, 4096, clr, 0)

        def p1(c, buf, carry):
            def inner(j, cc):
                v = buf[pl.ds(j * 16, 16)]
                u = plsc.bitcast(v, jnp.int32)
                b = lax.shift_right_arithmetic(_skey(u), 20) + 2048
                plsc.addupdate_scatter(hist, [lane4096 + b], ones)
                return cc

            return lax.fori_loop(0, VPC, inner, carry)

        stream_row(row, p1, 0)

        def scan_step(t, carry):
            cumtop, found, bstar = carry
            j = 255 - t
            acc = hist[pl.ds(j * 16, 16)]
            for l in range(1, 16):
                acc = acc + hist[pl.ds(l * 4096 + j * 16, 16)]
            cr = plsc.cumsum(jnp.flip(acc)) + cumtop
            cnt = s_at(plsc.all_reduce_population_count(cr >= K), 0)
            hit = jnp.logical_and(found == 0, cnt > 0)
            bstar = lax.select(hit, j * 16 + cnt - 1, bstar)
            found = lax.select(cnt > 0, jnp.int32(1), found)
            return splat(s_at(cr, 15)), found, bstar

        _, _, bstar = lax.fori_loop(
            0, 256, scan_step, (splat(jnp.int32(0)), jnp.int32(0), jnp.int32(0)))

        ts = lax.shift_left(bstar - 2048, 20)
        tm16 = splat(jnp.maximum(ts, _INT_MIN + 1024) - 1024)

        def p2(c, buf, cur):
            base_c = c * CHUNK

            def inner(j, cur):
                v = buf[pl.ds(j * 16, 16)]
                u = plsc.bitcast(v, jnp.int32)
                maskv = _skey(u) >= tm16
                cnt = s_at(plsc.all_reduce_population_count(maskv), 0)

                @pl.when(cnt > 0)
                def _():
                    off = jnp.minimum(cur, CKLEN - 16)
                    plsc.store_compressed(ck.at[pl.ds(off, 16)], u, mask=maskv)
                    plsc.store_compressed(
                        ci.at[pl.ds(off, 16)], iota + (base_c + j * 16),
                        mask=maskv)

                return cur + cnt

            return lax.fori_loop(0, VPC, inner, cur)

        ncand = stream_row(row, p2, jnp.int32(0))
        ncand16 = splat(ncand)

        def tf(j, c):
            xb = ck[pl.ds(j * 16, 16)]
            s = plsc.bitcast(xb, jnp.float32) / jnp.float32(0.8)
            uk = _skey(plsc.bitcast(s, jnp.int32)) ^ jnp.int32(0x7FFFFFFF)
            sel = (iota + j * 16) < ncand16
            ck[pl.ds(j * 16, 16)] = jnp.where(sel, uk, jnp.int32(-1))
            return c

        lax.fori_loop(0, CAP // 16, tf, 0)

        for p in range(4):
            src_k, src_i, dst_k, dst_i = (
                (ck, ci, ckb, cib) if p % 2 == 0 else (ckb, cib, ck, ci))
            sh = jnp.int32(8 * p)

            def rclr(j, c):
                h3[pl.ds(j * 16, 16)] = jnp.zeros((16,), jnp.int32)
                return c

            lax.fori_loop(0, 256, rclr, 0)

            def rhist(t, c, src_k=src_k, sh=sh):
                kv = plsc.load_gather(src_k, [lane128 + t])
                d = lax.shift_right_logical(kv, sh) & 255
                plsc.addupdate_scatter(h3, [d * 16 + iota], ones)
                return c

            lax.fori_loop(0, 128, rhist, 0)

            def rscan(j, carry):
                v = h3[pl.ds(j * 16, 16)]
                cinc = plsc.cumsum(v)
                h3[pl.ds(j * 16, 16)] = cinc - v + carry
                return carry + splat(s_at(cinc, 15))

            lax.fori_loop(0, 256, rscan, splat(jnp.int32(0)))

            def rscat(t, c, src_k=src_k, src_i=src_i, dst_k=dst_k,
                      dst_i=dst_i, sh=sh):
                idxv = lane128 + t
                kv = plsc.load_gather(src_k, [idxv])
                iv = plsc.load_gather(src_i, [idxv])
                addr = (lax.shift_right_logical(kv, sh) & 255) * 16 + iota
                pos = plsc.load_gather(h3, [addr])
                plsc.addupdate_scatter(h3, [addr], ones)
                plsc.store_scatter(dst_k, [pos], kv)
                plsc.store_scatter(dst_i, [pos], iv)
                return c

            lax.fori_loop(0, 128, rscat, 0)

        def emit(j, c):
            uk = ck[pl.ds(j * 16, 16)]
            sk = uk ^ jnp.int32(0x7FFFFFFF)
            su = sk ^ lax.shift_right_logical(
                lax.shift_right_arithmetic(sk, 31), 1)
            vstage[pl.ds(j * 16, 16)] = plsc.bitcast(su, jnp.float32)
            istage[pl.ds(j * 16, 16)] = ci[pl.ds(j * 16, 16)]
            return c

        lax.fori_loop(0, K // 16, emit, 0)
        pltpu.sync_copy(vstage, vals_out.at[row])
        pltpu.sync_copy(istage, idx_out.at[row])

    for r in range(ROWS_PER_W):
        do_row(wid * ROWS_PER_W + r)


def _make_topk(interpret=False):
  return functools.partial(
    pl.kernel,
    out_type=(jax.ShapeDtypeStruct((ROWS, K), jnp.float32),
              jax.ShapeDtypeStruct((ROWS, K), jnp.int32)),
    mesh=plsc.VectorSubcoreMesh(core_axis_name="c", subcore_axis_name="s",
                                num_cores=2, num_subcores=16),
    interpret=interpret,
    compiler_params=pltpu.CompilerParams(use_tc_tiling_on_sc=False,
                                         needs_layout_passes=False),
    scratch_types=[
        pltpu.VMEM((CHUNK,), jnp.float32),
        pltpu.VMEM((CHUNK,), jnp.float32),
        pltpu.VMEM((CKLEN,), jnp.int32),
        pltpu.VMEM((CKLEN,), jnp.int32),
        pltpu.VMEM((CAP,), jnp.int32),
        pltpu.VMEM((CAP,), jnp.int32),
        pltpu.VMEM((65536,), jnp.int32),
        pltpu.VMEM((4096,), jnp.int32),
        pltpu.VMEM((K,), jnp.float32),
        pltpu.VMEM((K,), jnp.int32),
        pltpu.SemaphoreType.DMA,
        pltpu.SemaphoreType.DMA,
    ],
  )(_topk_body)


_topk_sc = _make_topk()


def kernel(scores):
    vals_desc, indices = _topk_sc(scores)
    values = jnp.flip(vals_desc, axis=1)
    cumulative_probs = jnp.cumsum(jax.nn.softmax(values, axis=-1), axis=-1)
    sorted_indices_to_remove = cumulative_probs <= (1.0 - 0.9)
    values = jnp.where(sorted_indices_to_remove,
                      jnp.finfo(values.dtype).min, values)
    return (values, indices)


# speedup vs baseline: 4.0295x; 1.2448x over previous
import functools

import jax
import jax.numpy as jnp
from jax import lax
from jax.experimental import pallas as pl
from jax.experimental.pallas import tpu as pltpu
from jax.experimental.pallas import tpu_sc as plsc

ROWS = 64
N = 1_000_000
K = 1024
CHUNK = 10_000
NCHUNK = N // CHUNK
VPC = CHUNK // 16
CAP = 2048
CKLEN = CAP + 16
ROWS_PER_W = 2
_INT_MIN = -2147483648


def _skey(u):
    m = lax.shift_right_arithmetic(u, 31)
    return u ^ lax.shift_right_logical(m, 1)


def _topk_body(scores, vals_out, idx_out, buf0, buf1, ck, ci, ckb, cib,
               hist, h3, vstage, istage, sem0, sem1):
    iota = lax.iota(jnp.int32, 16)
    ones = jnp.ones((16,), jnp.int32)
    lane4096 = iota * 4096
    lane4096p = lane4096 + 2048
    lane128 = iota * 128
    UNROLL = 5
    NITER = VPC // UNROLL

    def s_at(v, i):
        return lax.squeeze(lax.slice(v, (i,), (i + 1,)), (0,))

    def splat(x):
        return lax.broadcast(x, (16,))

    wid = lax.axis_index("s") * 2 + lax.axis_index("c")

    def stream_row(row, process, carry0):
        pltpu.async_copy(scores.at[row, pl.ds(0, CHUNK)], buf0, sem0)

        def outer(i, carry):
            c0 = 2 * i
            cp1 = pltpu.async_copy(
                scores.at[row, pl.ds((c0 + 1) * CHUNK, CHUNK)], buf1, sem1)
            pltpu.make_async_copy(
                scores.at[row, pl.ds(c0 * CHUNK, CHUNK)], buf0, sem0).wait()
            carry = process(c0, buf0, carry)

            @pl.when(i < NCHUNK // 2 - 1)
            def _():
                pltpu.async_copy(
                    scores.at[row, pl.ds((c0 + 2) * CHUNK, CHUNK)], buf0, sem0)

            cp1.wait()
            carry = process(c0 + 1, buf1, carry)
            return carry

        return lax.fori_loop(0, NCHUNK // 2, outer, carry0)

    def do_row(row):
        zeros16 = jnp.zeros((16,), jnp.int32)

        def clr(j, c):
            for t in range(16):
                hist[pl.ds(j * 256 + t * 16, 16)] = zeros16
            return c

        lax.fori_loop(0, 256, clr, 0)

        def p1(c, buf, carry):
            def inner(j, cc):
                base = j * (16 * UNROLL)
                for t in range(UNROLL):
                    v = buf[pl.ds(base + t * 16, 16)]
                    u = plsc.bitcast(v, jnp.int32)
                    b = lax.shift_right_arithmetic(_skey(u), 20)
                    plsc.addupdate_scatter(hist, [lane4096p + b], ones)
                return cc

            return lax.fori_loop(0, NITER, inner, carry)

        stream_row(row, p1, 0)

        def scan_step(t, carry):
            cumtop, found, bstar = carry
            j = 255 - t
            acc = hist[pl.ds(j * 16, 16)]
            for l in range(1, 16):
                acc = acc + hist[pl.ds(l * 4096 + j * 16, 16)]
            cr = plsc.cumsum(jnp.flip(acc)) + cumtop
            cnt = s_at(plsc.all_reduce_population_count(cr >= K), 0)
            hit = jnp.logical_and(found == 0, cnt > 0)
            bstar = lax.select(hit, j * 16 + cnt - 1, bstar)
            found = lax.select(cnt > 0, jnp.int32(1), found)
            return splat(s_at(cr, 15)), found, bstar

        _, _, bstar = lax.fori_loop(
            0, 256, scan_step, (splat(jnp.int32(0)), jnp.int32(0), jnp.int32(0)))

        ts = lax.shift_left(bstar - 2048, 20)
        tm16 = splat(jnp.maximum(ts, _INT_MIN + 1024) - 1024)

        def p2(c, buf, cur):
            base_c = c * CHUNK

            def inner(j, cur):
                base = j * (16 * UNROLL)
                us, masks, cnts = [], [], []
                total = jnp.int32(0)
                for t in range(UNROLL):
                    v = buf[pl.ds(base + t * 16, 16)]
                    u = plsc.bitcast(v, jnp.int32)
                    maskv = _skey(u) >= tm16
                    cnt = s_at(plsc.all_reduce_population_count(maskv), 0)
                    us.append(u)
                    masks.append(maskv)
                    cnts.append(cnt)
                    total = total + cnt

                @pl.when(total > 0)
                def _():
                    off = cur
                    for t in range(UNROLL):
                        o = jnp.minimum(off, CKLEN - 16)
                        plsc.store_compressed(
                            ck.at[pl.ds(o, 16)], us[t], mask=masks[t])
                        plsc.store_compressed(
                            ci.at[pl.ds(o, 16)],
                            iota + (base_c + base + t * 16), mask=masks[t])
                        off = off + cnts[t]

                return cur + total

            return lax.fori_loop(0, NITER, inner, cur)

        ncand = stream_row(row, p2, jnp.int32(0))
        ncand16 = splat(ncand)

        def tf(j, c):
            xb = ck[pl.ds(j * 16, 16)]
            s = plsc.bitcast(xb, jnp.float32) / jnp.float32(0.8)
            uk = _skey(plsc.bitcast(s, jnp.int32)) ^ jnp.int32(0x7FFFFFFF)
            sel = (iota + j * 16) < ncand16
            ck[pl.ds(j * 16, 16)] = jnp.where(sel, uk, jnp.int32(-1))
            return c

        lax.fori_loop(0, CAP // 16, tf, 0)

        for p in range(4):
            src_k, src_i, dst_k, dst_i = (
                (ck, ci, ckb, cib) if p % 2 == 0 else (ckb, cib, ck, ci))
            sh = jnp.int32(8 * p)

            def rclr(j, c):
                for t in range(16):
                    h3[pl.ds(j * 256 + t * 16, 16)] = jnp.zeros((16,), jnp.int32)
                return c

            lax.fori_loop(0, 16, rclr, 0)

            def rhist(t4, c, src_k=src_k, sh=sh):
                for t in range(4):
                    kv = plsc.load_gather(src_k, [lane128 + (t4 * 4 + t)])
                    d = lax.shift_right_logical(kv, sh) & 255
                    plsc.addupdate_scatter(h3, [d * 16 + iota], ones)
                return c

            lax.fori_loop(0, 32, rhist, 0)

            def rscan(j, carry):
                v = h3[pl.ds(j * 16, 16)]
                cinc = plsc.cumsum(v)
                h3[pl.ds(j * 16, 16)] = cinc - v + carry
                return carry + splat(s_at(cinc, 15))

            lax.fori_loop(0, 256, rscan, splat(jnp.int32(0)))

            def rscat(t4, c, src_k=src_k, src_i=src_i, dst_k=dst_k,
                      dst_i=dst_i, sh=sh):
                for t in range(4):
                    idxv = lane128 + (t4 * 4 + t)
                    kv = plsc.load_gather(src_k, [idxv])
                    iv = plsc.load_gather(src_i, [idxv])
                    addr = (lax.shift_right_logical(kv, sh) & 255) * 16 + iota
                    pos = plsc.load_gather(h3, [addr])
                    plsc.addupdate_scatter(h3, [addr], ones)
                    plsc.store_scatter(dst_k, [pos], kv)
                    plsc.store_scatter(dst_i, [pos], iv)
                return c

            lax.fori_loop(0, 32, rscat, 0)

        def emit(j, c):
            uk = ck[pl.ds(j * 16, 16)]
            sk = uk ^ jnp.int32(0x7FFFFFFF)
            su = sk ^ lax.shift_right_logical(
                lax.shift_right_arithmetic(sk, 31), 1)
            vstage[pl.ds(j * 16, 16)] = plsc.bitcast(su, jnp.float32)
            istage[pl.ds(j * 16, 16)] = ci[pl.ds(j * 16, 16)]
            return c

        lax.fori_loop(0, K // 16, emit, 0)
        pltpu.sync_copy(vstage, vals_out.at[row])
        pltpu.sync_copy(istage, idx_out.at[row])

    for r in range(ROWS_PER_W):
        do_row(wid * ROWS_PER_W + r)


def _make_topk(interpret=False):
  return functools.partial(
    pl.kernel,
    out_type=(jax.ShapeDtypeStruct((ROWS, K), jnp.float32),
              jax.ShapeDtypeStruct((ROWS, K), jnp.int32)),
    mesh=plsc.VectorSubcoreMesh(core_axis_name="c", subcore_axis_name="s",
                                num_cores=2, num_subcores=16),
    interpret=interpret,
    compiler_params=pltpu.CompilerParams(use_tc_tiling_on_sc=False,
                                         needs_layout_passes=False),
    scratch_types=[
        pltpu.VMEM((CHUNK,), jnp.float32),
        pltpu.VMEM((CHUNK,), jnp.float32),
        pltpu.VMEM((CKLEN,), jnp.int32),
        pltpu.VMEM((CKLEN,), jnp.int32),
        pltpu.VMEM((CAP,), jnp.int32),
        pltpu.VMEM((CAP,), jnp.int32),
        pltpu.VMEM((65536,), jnp.int32),
        pltpu.VMEM((4096,), jnp.int32),
        pltpu.VMEM((K,), jnp.float32),
        pltpu.VMEM((K,), jnp.int32),
        pltpu.SemaphoreType.DMA,
        pltpu.SemaphoreType.DMA,
    ],
  )(_topk_body)


_topk_sc = _make_topk()


def kernel(scores):
    vals_desc, indices = _topk_sc(scores)
    values = jnp.flip(vals_desc, axis=1)
    cumulative_probs = jnp.cumsum(jax.nn.softmax(values, axis=-1), axis=-1)
    sorted_indices_to_remove = cumulative_probs <= (1.0 - 0.9)
    values = jnp.where(sorted_indices_to_remove,
                      jnp.finfo(values.dtype).min, values)
    return (values, indices)


# speedup vs baseline: 4.2459x; 1.0537x over previous
import functools

import jax
import jax.numpy as jnp
from jax import lax
from jax.experimental import pallas as pl
from jax.experimental.pallas import tpu as pltpu
from jax.experimental.pallas import tpu_sc as plsc

ROWS = 64
N = 1_000_000
K = 1024
CHUNK = 10_000
NCHUNK = N // CHUNK
VPC = CHUNK // 16
CAP = 2048
CKLEN = CAP + 16
ROWS_PER_W = 2
_INT_MIN = -2147483648


def _skey(u):
    m = lax.shift_right_arithmetic(u, 31)
    return u ^ lax.shift_right_logical(m, 1)


def _topk_body(scores, vals_out, idx_out, buf0, buf1, ck, ci, ckb, cib,
               hist, h3, vstage, istage, sem0, sem1):
    iota = lax.iota(jnp.int32, 16)
    ones = jnp.ones((16,), jnp.int32)
    lane4096 = iota * 4096
    lane4096p = lane4096 + 2048
    lane128 = iota * 128
    UNROLL = 25
    NITER = VPC // UNROLL

    def s_at(v, i):
        return lax.squeeze(lax.slice(v, (i,), (i + 1,)), (0,))

    def splat(x):
        return lax.broadcast(x, (16,))

    wid = lax.axis_index("s") * 2 + lax.axis_index("c")

    def stream_row(row, process, carry0):
        pltpu.async_copy(scores.at[row, pl.ds(0, CHUNK)], buf0, sem0)

        def outer(i, carry):
            c0 = 2 * i
            cp1 = pltpu.async_copy(
                scores.at[row, pl.ds((c0 + 1) * CHUNK, CHUNK)], buf1, sem1)
            pltpu.make_async_copy(
                scores.at[row, pl.ds(c0 * CHUNK, CHUNK)], buf0, sem0).wait()
            carry = process(c0, buf0, carry)

            @pl.when(i < NCHUNK // 2 - 1)
            def _():
                pltpu.async_copy(
                    scores.at[row, pl.ds((c0 + 2) * CHUNK, CHUNK)], buf0, sem0)

            cp1.wait()
            carry = process(c0 + 1, buf1, carry)
            return carry

        return lax.fori_loop(0, NCHUNK // 2, outer, carry0)

    def do_row(row):
        zeros16 = jnp.zeros((16,), jnp.int32)

        def clr(j, c):
            for t in range(16):
                hist[pl.ds(j * 256 + t * 16, 16)] = zeros16
            return c

        lax.fori_loop(0, 256, clr, 0)

        def p1(c, buf, carry):
            def inner(j, cc):
                base = j * (16 * UNROLL)
                for t in range(UNROLL):
                    v = buf[pl.ds(base + t * 16, 16)]
                    u = plsc.bitcast(v, jnp.int32)
                    b = lax.shift_right_arithmetic(_skey(u), 20)
                    plsc.addupdate_scatter(hist, [lane4096p + b], ones)
                return cc

            return lax.fori_loop(0, NITER, inner, carry)

        stream_row(row, p1, 0)

        def scan_step(t, carry):
            cumtop, found, bstar = carry
            j = 255 - t
            acc = hist[pl.ds(j * 16, 16)]
            for l in range(1, 16):
                acc = acc + hist[pl.ds(l * 4096 + j * 16, 16)]
            cr = plsc.cumsum(jnp.flip(acc)) + cumtop
            cnt = s_at(plsc.all_reduce_population_count(cr >= K), 0)
            hit = jnp.logical_and(found == 0, cnt > 0)
            bstar = lax.select(hit, j * 16 + cnt - 1, bstar)
            found = lax.select(cnt > 0, jnp.int32(1), found)
            return splat(s_at(cr, 15)), found, bstar

        _, _, bstar = lax.fori_loop(
            0, 256, scan_step, (splat(jnp.int32(0)), jnp.int32(0), jnp.int32(0)))

        ts = lax.shift_left(bstar - 2048, 20)
        tm16 = splat(jnp.maximum(ts, _INT_MIN + 1024) - 1024)

        def p2(c, buf, cur):
            base_c = c * CHUNK

            def inner(j, cur):
                base = j * (16 * UNROLL)
                us, masks, cnts = [], [], []
                total = jnp.int32(0)
                for t in range(UNROLL):
                    v = buf[pl.ds(base + t * 16, 16)]
                    u = plsc.bitcast(v, jnp.int32)
                    maskv = _skey(u) >= tm16
                    cnt = s_at(plsc.all_reduce_population_count(maskv), 0)
                    us.append(u)
                    masks.append(maskv)
                    cnts.append(cnt)
                    total = total + cnt

                @pl.when(total > 0)
                def _():
                    off = cur
                    for t in range(UNROLL):
                        o = jnp.minimum(off, CKLEN - 16)
                        plsc.store_compressed(
                            ck.at[pl.ds(o, 16)], us[t], mask=masks[t])
                        plsc.store_compressed(
                            ci.at[pl.ds(o, 16)],
                            iota + (base_c + base + t * 16), mask=masks[t])
                        off = off + cnts[t]

                return cur + total

            return lax.fori_loop(0, NITER, inner, cur)

        ncand = stream_row(row, p2, jnp.int32(0))
        ncand16 = splat(ncand)

        def tf(j, c):
            xb = ck[pl.ds(j * 16, 16)]
            s = plsc.bitcast(xb, jnp.float32) / jnp.float32(0.8)
            uk = _skey(plsc.bitcast(s, jnp.int32)) ^ jnp.int32(0x7FFFFFFF)
            sel = (iota + j * 16) < ncand16
            ck[pl.ds(j * 16, 16)] = jnp.where(sel, uk, jnp.int32(-1))
            return c

        lax.fori_loop(0, CAP // 16, tf, 0)

        for p in range(4):
            src_k, src_i, dst_k, dst_i = (
                (ck, ci, ckb, cib) if p % 2 == 0 else (ckb, cib, ck, ci))
            sh = jnp.int32(8 * p)

            def rclr(j, c):
                for t in range(16):
                    h3[pl.ds(j * 256 + t * 16, 16)] = jnp.zeros((16,), jnp.int32)
                return c

            lax.fori_loop(0, 16, rclr, 0)

            def rhist(t4, c, src_k=src_k, sh=sh):
                for t in range(4):
                    kv = plsc.load_gather(src_k, [lane128 + (t4 * 4 + t)])
                    d = lax.shift_right_logical(kv, sh) & 255
                    plsc.addupdate_scatter(h3, [d * 16 + iota], ones)
                return c

            lax.fori_loop(0, 32, rhist, 0)

            def rscan(j, carry):
                v = h3[pl.ds(j * 16, 16)]
                cinc = plsc.cumsum(v)
                h3[pl.ds(j * 16, 16)] = cinc - v + carry
                return carry + splat(s_at(cinc, 15))

            lax.fori_loop(0, 256, rscan, splat(jnp.int32(0)))

            def rscat(t4, c, src_k=src_k, src_i=src_i, dst_k=dst_k,
                      dst_i=dst_i, sh=sh):
                for t in range(4):
                    idxv = lane128 + (t4 * 4 + t)
                    kv = plsc.load_gather(src_k, [idxv])
                    iv = plsc.load_gather(src_i, [idxv])
                    addr = (lax.shift_right_logical(kv, sh) & 255) * 16 + iota
                    pos = plsc.load_gather(h3, [addr])
                    plsc.addupdate_scatter(h3, [addr], ones)
                    plsc.store_scatter(dst_k, [pos], kv)
                    plsc.store_scatter(dst_i, [pos], iv)
                return c

            lax.fori_loop(0, 32, rscat, 0)

        def emit(j, c):
            uk = ck[pl.ds(j * 16, 16)]
            sk = uk ^ jnp.int32(0x7FFFFFFF)
            su = sk ^ lax.shift_right_logical(
                lax.shift_right_arithmetic(sk, 31), 1)
            vstage[pl.ds(j * 16, 16)] = plsc.bitcast(su, jnp.float32)
            istage[pl.ds(j * 16, 16)] = ci[pl.ds(j * 16, 16)]
            return c

        lax.fori_loop(0, K // 16, emit, 0)
        pltpu.sync_copy(vstage, vals_out.at[row])
        pltpu.sync_copy(istage, idx_out.at[row])

    for r in range(ROWS_PER_W):
        do_row(wid * ROWS_PER_W + r)


def _make_topk(interpret=False):
  return functools.partial(
    pl.kernel,
    out_type=(jax.ShapeDtypeStruct((ROWS, K), jnp.float32),
              jax.ShapeDtypeStruct((ROWS, K), jnp.int32)),
    mesh=plsc.VectorSubcoreMesh(core_axis_name="c", subcore_axis_name="s",
                                num_cores=2, num_subcores=16),
    interpret=interpret,
    compiler_params=pltpu.CompilerParams(use_tc_tiling_on_sc=False,
                                         needs_layout_passes=False),
    scratch_types=[
        pltpu.VMEM((CHUNK,), jnp.float32),
        pltpu.VMEM((CHUNK,), jnp.float32),
        pltpu.VMEM((CKLEN,), jnp.int32),
        pltpu.VMEM((CKLEN,), jnp.int32),
        pltpu.VMEM((CAP,), jnp.int32),
        pltpu.VMEM((CAP,), jnp.int32),
        pltpu.VMEM((65536,), jnp.int32),
        pltpu.VMEM((4096,), jnp.int32),
        pltpu.VMEM((K,), jnp.float32),
        pltpu.VMEM((K,), jnp.int32),
        pltpu.SemaphoreType.DMA,
        pltpu.SemaphoreType.DMA,
    ],
  )(_topk_body)


_topk_sc = _make_topk()


def kernel(scores):
    vals_desc, indices = _topk_sc(scores)
    values = jnp.flip(vals_desc, axis=1)
    cumulative_probs = jnp.cumsum(jax.nn.softmax(values, axis=-1), axis=-1)
    sorted_indices_to_remove = cumulative_probs <= (1.0 - 0.9)
    values = jnp.where(sorted_indices_to_remove,
                      jnp.finfo(values.dtype).min, values)
    return (values, indices)
